# Initial kernel scaffold; baseline (speedup 1.0000x reference)
import jax
import jax.numpy as jnp
from jax.experimental import pallas as pl


def kernel(obs, table):
    raise NotImplementedError("write your pallas kernel here")


# speedup vs baseline: 1.4170x; 1.4170x over previous
import functools

import jax
import jax.numpy as jnp
from jax import lax
from jax.experimental import pallas as pl
from jax.experimental.pallas import tpu as pltpu
from jax.experimental.pallas import tpu_sc as plsc

NUM_STATES = 16
NUM_ACTIONS = 8
BATCH = 16384
OBS_DIM = 4

NC = 2
NS = 16
L = 16
NW = NC * NS
B_PER_W = BATCH // NW
GROUPS = B_PER_W // L


def _body(obs_hbm, table_hbm, out_hbm, obs_v, tab_v, out_v):
    wid = lax.axis_index("s") * NC + lax.axis_index("c")
    base = wid * B_PER_W

    pltpu.sync_copy(obs_hbm.at[pl.ds(base, B_PER_W)], obs_v)
    pltpu.sync_copy(table_hbm, tab_v)

    lanes = lax.iota(jnp.int32, L)

    def group(g, _):
        rows = g * L + lanes
        states = plsc.load_gather(obs_v, [rows, jnp.zeros((L,), jnp.int32)])
        for j in range(NUM_ACTIONS):
            col = jnp.full((L,), j, jnp.int32)
            vals = plsc.load_gather(tab_v, [states, col])
            plsc.store_scatter(out_v, [rows, col], vals)
        return _

    lax.fori_loop(0, GROUPS, group, 0, unroll=4)

    pltpu.sync_copy(out_v, out_hbm.at[pl.ds(base, B_PER_W)])


@jax.jit
def kernel(obs, table):
    mesh = plsc.VectorSubcoreMesh(
        core_axis_name="c", subcore_axis_name="s", num_cores=NC, num_subcores=NS
    )
    k = pl.kernel(
        _body,
        out_type=jax.ShapeDtypeStruct((BATCH, NUM_ACTIONS), jnp.float32),
        mesh=mesh,
        scratch_types=[
            pltpu.VMEM((B_PER_W, OBS_DIM), jnp.int32),
            pltpu.VMEM((NUM_STATES, NUM_ACTIONS), jnp.float32),
            pltpu.VMEM((B_PER_W, NUM_ACTIONS), jnp.float32),
        ],
        compiler_params=pltpu.CompilerParams(
            needs_layout_passes=False, use_tc_tiling_on_sc=False
        ),
    )
    return k(obs, table)


# speedup vs baseline: 2.0789x; 1.4671x over previous
import jax
import jax.numpy as jnp
from jax import lax
from jax.experimental import pallas as pl
from jax.experimental.pallas import tpu as pltpu
from jax.experimental.pallas import tpu_sc as plsc

NUM_STATES = 16
NUM_ACTIONS = 8
BATCH = 16384

NC = 2
NS = 16
L = 16
NW = NC * NS
B_PER_W = BATCH // NW
GROUPS = B_PER_W // L


def _body(states_hbm, table_hbm, out_hbm, idx_v, tab_v, out_v, sem1, sem2):
    wid = lax.axis_index("s") * NC + lax.axis_index("c")
    base = wid * B_PER_W

    c1 = pltpu.make_async_copy(states_hbm.at[pl.ds(base, B_PER_W)], idx_v, sem1)
    c2 = pltpu.make_async_copy(table_hbm, tab_v, sem2)
    c1.start()
    c2.start()
    c1.wait()
    c2.wait()

    lanes = lax.iota(jnp.int32, L)

    def group(g, _):
        rows = g * L + lanes
        states = idx_v[pl.ds(g * L, L)]
        for j in range(NUM_ACTIONS):
            col = jnp.full((L,), j, jnp.int32)
            vals = plsc.load_gather(tab_v, [states, col])
            plsc.store_scatter(out_v, [rows, col], vals)
        return _

    lax.fori_loop(0, GROUPS, group, 0, unroll=4)

    pltpu.sync_copy(out_v, out_hbm.at[pl.ds(base, B_PER_W)])


_mesh = plsc.VectorSubcoreMesh(
    core_axis_name="c", subcore_axis_name="s", num_cores=NC, num_subcores=NS
)
_lookup = pl.kernel(
    _body,
    out_type=jax.ShapeDtypeStruct((BATCH, NUM_ACTIONS), jnp.float32),
    mesh=_mesh,
    scratch_types=[
        pltpu.VMEM((B_PER_W,), jnp.int32),
        pltpu.VMEM((NUM_STATES, NUM_ACTIONS), jnp.float32),
        pltpu.VMEM((B_PER_W, NUM_ACTIONS), jnp.float32),
        pltpu.SemaphoreType.DMA,
        pltpu.SemaphoreType.DMA,
    ],
    compiler_params=pltpu.CompilerParams(
        needs_layout_passes=False, use_tc_tiling_on_sc=False
    ),
)


@jax.jit
def kernel(obs, table):
    states = obs[:, 0]
    return _lookup(states, table)


# speedup vs baseline: 3.2531x; 1.5648x over previous
import jax
import jax.numpy as jnp
from jax import lax
from jax.experimental import pallas as pl
from jax.experimental.pallas import tpu as pltpu
from jax.experimental.pallas import tpu_sc as plsc

NUM_STATES = 16
NUM_ACTIONS = 8
BATCH = 16384

NC = 2
NS = 16
L = 16
NW = NC * NS
B_PER_W = BATCH // NW
CBLK = BATCH // 128
CB_PER_W = B_PER_W // 128
GROUPS = B_PER_W // L


def _body(states_hbm, table_hbm, out_hbm, idx_v, tab_v, out_v, sem1, sem2):
    wid = lax.axis_index("s") * NC + lax.axis_index("c")
    base = wid * B_PER_W

    c1 = pltpu.make_async_copy(states_hbm.at[pl.ds(base, B_PER_W)], idx_v, sem1)
    c2 = pltpu.make_async_copy(table_hbm, tab_v, sem2)
    c1.start()
    c2.start()
    c1.wait()
    c2.wait()

    def group(g, _):
        states = idx_v[pl.ds(g * L, L)]
        s8 = states * NUM_ACTIONS
        cb = g // (128 // L)
        lo = (g % (128 // L)) * L
        for j in range(NUM_ACTIONS):
            vals = plsc.load_gather(tab_v, [s8 + j])
            out_v[cb, j, pl.ds(lo, L)] = vals
        return _

    lax.fori_loop(0, GROUPS, group, 0, unroll=8)

    pltpu.sync_copy(out_v, out_hbm.at[pl.ds(wid * CB_PER_W, CB_PER_W)])


_mesh = plsc.VectorSubcoreMesh(
    core_axis_name="c", subcore_axis_name="s", num_cores=NC, num_subcores=NS
)
_lookup = pl.kernel(
    _body,
    out_type=jax.ShapeDtypeStruct((CBLK, NUM_ACTIONS, 128), jnp.float32),
    mesh=_mesh,
    scratch_types=[
        pltpu.VMEM((B_PER_W,), jnp.int32),
        pltpu.VMEM((NUM_STATES * NUM_ACTIONS,), jnp.float32),
        pltpu.VMEM((CB_PER_W, NUM_ACTIONS, 128), jnp.float32),
        pltpu.SemaphoreType.DMA,
        pltpu.SemaphoreType.DMA,
    ],
    compiler_params=pltpu.CompilerParams(
        needs_layout_passes=False, use_tc_tiling_on_sc=False
    ),
)


@jax.jit
def kernel(obs, table):
    states = obs[:, 0]
    a = _lookup(states, table.reshape(NUM_STATES * NUM_ACTIONS))
    return a.transpose(1, 0, 2).reshape(NUM_ACTIONS, BATCH).T


# speedup vs baseline: 3.4756x; 1.0684x over previous
import jax
import jax.numpy as jnp
from jax import lax
from jax.experimental import pallas as pl
from jax.experimental.pallas import tpu as pltpu
from jax.experimental.pallas import tpu_sc as plsc

NUM_STATES = 16
NUM_ACTIONS = 8
BATCH = 16384

NC = 2
NS = 16
L = 16
NW = NC * NS
B_PER_W = BATCH // NW
CBLK = BATCH // 128
CB_PER_W = B_PER_W // 128
GROUPS = B_PER_W // L


def _body(states_hbm, table_hbm, out_hbm, idx_v, tab_v, out_v, sem1, sem2):
    wid = lax.axis_index("s") * NC + lax.axis_index("c")
    base = wid * B_PER_W

    c1 = pltpu.make_async_copy(states_hbm.at[pl.ds(base, B_PER_W)], idx_v, sem1)
    c2 = pltpu.make_async_copy(table_hbm, tab_v, sem2)
    c1.start()
    c2.start()
    c1.wait()
    c2.wait()

    cols = [tab_v[pl.ds(j * NUM_STATES, NUM_STATES)] for j in range(NUM_ACTIONS)]

    for g in range(GROUPS):
        states = idx_v[pl.ds(g * L, L)]
        cb = g // (128 // L)
        lo = (g % (128 // L)) * L
        for j in range(NUM_ACTIONS):
            out_v[cb, j, pl.ds(lo, L)] = jnp.take_along_axis(
                cols[j], states, axis=0
            )

    pltpu.sync_copy(out_v, out_hbm.at[pl.ds(wid * CB_PER_W, CB_PER_W)])


_mesh = plsc.VectorSubcoreMesh(
    core_axis_name="c", subcore_axis_name="s", num_cores=NC, num_subcores=NS
)
_lookup = pl.kernel(
    _body,
    out_type=jax.ShapeDtypeStruct((CBLK, NUM_ACTIONS, 128), jnp.float32),
    mesh=_mesh,
    scratch_types=[
        pltpu.VMEM((B_PER_W,), jnp.int32),
        pltpu.VMEM((NUM_STATES * NUM_ACTIONS,), jnp.float32),
        pltpu.VMEM((CB_PER_W, NUM_ACTIONS, 128), jnp.float32),
        pltpu.SemaphoreType.DMA,
        pltpu.SemaphoreType.DMA,
    ],
    compiler_params=pltpu.CompilerParams(
        needs_layout_passes=False, use_tc_tiling_on_sc=False
    ),
)


@jax.jit
def kernel(obs, table):
    states = obs[:, 0]
    a = _lookup(states, table.T.reshape(NUM_STATES * NUM_ACTIONS))
    return a.transpose(1, 0, 2).reshape(NUM_ACTIONS, BATCH).T


# speedup vs baseline: 3.5407x; 1.0188x over previous
import jax
import jax.numpy as jnp
from jax import lax
from jax.experimental import pallas as pl
from jax.experimental.pallas import tpu as pltpu
from jax.experimental.pallas import tpu_sc as plsc

NUM_STATES = 16
NUM_ACTIONS = 8
BATCH = 16384
OBS_DIM = 4

NC = 2
NS = 16
L = 16
NW = NC * NS
B_PER_W = BATCH // NW
CBLK = BATCH // 128
CB_PER_W = B_PER_W // 128
GROUPS = B_PER_W // L


def _body(obs_hbm, table_hbm, out_hbm, idx_v, tab_v, out_v, sem1, sem2):
    wid = lax.axis_index("s") * NC + lax.axis_index("c")

    c1 = pltpu.make_async_copy(
        obs_hbm.at[pl.ds(wid * CB_PER_W, CB_PER_W), pl.ds(0, 1)], idx_v, sem1
    )
    c2 = pltpu.make_async_copy(table_hbm, tab_v, sem2)
    c1.start()
    c2.start()
    c2.wait()

    lanes = lax.iota(jnp.int32, L)
    cols = [plsc.load_gather(tab_v, [lanes * NUM_ACTIONS + j])
            for j in range(NUM_ACTIONS)]
    c1.wait()

    for g in range(GROUPS):
        cb = g // (128 // L)
        lo = (g % (128 // L)) * L
        states = idx_v[cb, 0, pl.ds(lo, L)]
        for j in range(NUM_ACTIONS):
            out_v[cb, j, pl.ds(lo, L)] = jnp.take_along_axis(
                cols[j], states, axis=0
            )

    pltpu.sync_copy(out_v, out_hbm.at[pl.ds(wid * CB_PER_W, CB_PER_W)])


_mesh = plsc.VectorSubcoreMesh(
    core_axis_name="c", subcore_axis_name="s", num_cores=NC, num_subcores=NS
)
_lookup = pl.kernel(
    _body,
    out_type=jax.ShapeDtypeStruct((CBLK, NUM_ACTIONS, 128), jnp.float32),
    mesh=_mesh,
    scratch_types=[
        pltpu.VMEM((CB_PER_W, 1, 128), jnp.int32),
        pltpu.VMEM((NUM_STATES * NUM_ACTIONS,), jnp.float32),
        pltpu.VMEM((CB_PER_W, NUM_ACTIONS, 128), jnp.float32),
        pltpu.SemaphoreType.DMA,
        pltpu.SemaphoreType.DMA,
    ],
    compiler_params=pltpu.CompilerParams(
        needs_layout_passes=False, use_tc_tiling_on_sc=False
    ),
)


@jax.jit
def kernel(obs, table):
    obs_view = obs.T.reshape(OBS_DIM, CBLK, 128).transpose(1, 0, 2)
    a = _lookup(obs_view, table.reshape(NUM_STATES * NUM_ACTIONS))
    return a.transpose(1, 0, 2).reshape(NUM_ACTIONS, BATCH).T


# speedup vs baseline: 3.5720x; 1.0088x over previous
import jax
import jax.numpy as jnp
from jax import lax
from jax.experimental import pallas as pl
from jax.experimental.pallas import tpu as pltpu
from jax.experimental.pallas import tpu_sc as plsc

NUM_STATES = 16
NUM_ACTIONS = 8
BATCH = 16384
OBS_DIM = 4

NC = 2
NS = 16
L = 16
NW = NC * NS
B_PER_W = BATCH // NW
CBLK = BATCH // 128
CB_PER_W = B_PER_W // 128
GROUPS = B_PER_W // L


def _body(obs_hbm, table_hbm, out_hbm, idx_v, tab_v, out_v, sem1, sem2):
    wid = lax.axis_index("s") * NC + lax.axis_index("c")

    c1 = pltpu.make_async_copy(
        obs_hbm.at[pl.ds(wid * CB_PER_W, CB_PER_W), pl.ds(0, 1)], idx_v, sem1
    )
    c2 = pltpu.make_async_copy(table_hbm, tab_v, sem2)
    c1.start()
    c2.start()
    c2.wait()

    lanes = lax.iota(jnp.int32, L)
    cols = [plsc.load_gather(tab_v, [lanes * NUM_ACTIONS + j])
            for j in range(NUM_ACTIONS)]
    c1.wait()

    def group(g, _):
        cb = g // (128 // L)
        lo = (g % (128 // L)) * L
        states = idx_v[cb, 0, pl.ds(lo, L)]
        for j in range(NUM_ACTIONS):
            out_v[cb, j, pl.ds(lo, L)] = jnp.take_along_axis(
                cols[j], states, axis=0
            )
        return _

    lax.fori_loop(0, GROUPS, group, 0, unroll=2)

    pltpu.sync_copy(out_v, out_hbm.at[pl.ds(wid * CB_PER_W, CB_PER_W)])


_mesh = plsc.VectorSubcoreMesh(
    core_axis_name="c", subcore_axis_name="s", num_cores=NC, num_subcores=NS
)
_lookup = pl.kernel(
    _body,
    out_type=jax.ShapeDtypeStruct((CBLK, NUM_ACTIONS, 128), jnp.float32),
    mesh=_mesh,
    scratch_types=[
        pltpu.VMEM((CB_PER_W, 1, 128), jnp.int32),
        pltpu.VMEM((NUM_STATES * NUM_ACTIONS,), jnp.float32),
        pltpu.VMEM((CB_PER_W, NUM_ACTIONS, 128), jnp.float32),
        pltpu.SemaphoreType.DMA,
        pltpu.SemaphoreType.DMA,
    ],
    compiler_params=pltpu.CompilerParams(
        needs_layout_passes=False, use_tc_tiling_on_sc=False
    ),
)


@jax.jit
def kernel(obs, table):
    obs_view = obs.T.reshape(OBS_DIM, CBLK, 128).transpose(1, 0, 2)
    a = _lookup(obs_view, table.reshape(NUM_STATES * NUM_ACTIONS))
    return a.transpose(1, 0, 2).reshape(NUM_ACTIONS, BATCH).T


# speedup vs baseline: 3.8506x; 1.0780x over previous
import jax
import jax.numpy as jnp
from jax import lax
from jax.experimental import pallas as pl
from jax.experimental.pallas import tpu as pltpu
from jax.experimental.pallas import tpu_sc as plsc

NUM_STATES = 16
NUM_ACTIONS = 8
BATCH = 16384
OBS_DIM = 4

NC = 1
NS = 16
L = 16
NW = NC * NS
B_PER_W = BATCH // NW
CBLK = BATCH // 128
CB_PER_W = B_PER_W // 128
GROUPS = B_PER_W // L


def _body(obs_hbm, table_hbm, out_hbm, idx_v, tab_v, out_v, sem1, sem2):
    wid = lax.axis_index("s") * NC + lax.axis_index("c")

    c1 = pltpu.make_async_copy(
        obs_hbm.at[pl.ds(wid * CB_PER_W, CB_PER_W), pl.ds(0, 1)], idx_v, sem1
    )
    c2 = pltpu.make_async_copy(table_hbm, tab_v, sem2)
    c1.start()
    c2.start()
    c2.wait()

    lanes = lax.iota(jnp.int32, L)
    cols = [plsc.load_gather(tab_v, [lanes * NUM_ACTIONS + j])
            for j in range(NUM_ACTIONS)]
    c1.wait()

    def group(g, _):
        cb = g // (128 // L)
        lo = (g % (128 // L)) * L
        states = idx_v[cb, 0, pl.ds(lo, L)]
        for j in range(NUM_ACTIONS):
            out_v[cb, j, pl.ds(lo, L)] = jnp.take_along_axis(
                cols[j], states, axis=0
            )
        return _

    lax.fori_loop(0, GROUPS, group, 0, unroll=2)

    pltpu.sync_copy(out_v, out_hbm.at[pl.ds(wid * CB_PER_W, CB_PER_W)])


_mesh = plsc.VectorSubcoreMesh(
    core_axis_name="c", subcore_axis_name="s", num_cores=NC, num_subcores=NS
)
_lookup = pl.kernel(
    _body,
    out_type=jax.ShapeDtypeStruct((CBLK, NUM_ACTIONS, 128), jnp.float32),
    mesh=_mesh,
    scratch_types=[
        pltpu.VMEM((CB_PER_W, 1, 128), jnp.int32),
        pltpu.VMEM((NUM_STATES * NUM_ACTIONS,), jnp.float32),
        pltpu.VMEM((CB_PER_W, NUM_ACTIONS, 128), jnp.float32),
        pltpu.SemaphoreType.DMA,
        pltpu.SemaphoreType.DMA,
    ],
    compiler_params=pltpu.CompilerParams(
        needs_layout_passes=False, use_tc_tiling_on_sc=False
    ),
)


@jax.jit
def kernel(obs, table):
    obs_view = obs.T.reshape(OBS_DIM, CBLK, 128).transpose(1, 0, 2)
    a = _lookup(obs_view, table.reshape(NUM_STATES * NUM_ACTIONS))
    return a.transpose(1, 0, 2).reshape(NUM_ACTIONS, BATCH).T


# speedup vs baseline: 3.8602x; 1.0025x over previous
import jax
import jax.numpy as jnp
from jax import lax
from jax.experimental import pallas as pl
from jax.experimental.pallas import tpu as pltpu
from jax.experimental.pallas import tpu_sc as plsc

NUM_STATES = 16
NUM_ACTIONS = 8
BATCH = 16384
OBS_DIM = 4

NC = 1
NS = 16
L = 16
NW = NC * NS
B_PER_W = BATCH // NW
CBLK = BATCH // 128
CB_PER_W = B_PER_W // 128
GROUPS = B_PER_W // L


def _body(obs_hbm, table_hbm, out_hbm, idx_v, tab_v, out_v, sem1, sem2):
    wid = lax.axis_index("s") * NC + lax.axis_index("c")

    c1 = pltpu.make_async_copy(
        obs_hbm.at[pl.ds(wid * CB_PER_W, CB_PER_W), pl.ds(0, 1)], idx_v, sem1
    )
    c2 = pltpu.make_async_copy(table_hbm, tab_v, sem2)
    c1.start()
    c2.start()
    c2.wait()

    lanes = lax.iota(jnp.int32, L)
    cols = [plsc.load_gather(tab_v, [lanes * NUM_ACTIONS + j])
            for j in range(NUM_ACTIONS)]
    c1.wait()

    def group(g, _):
        cb = g // (128 // L)
        lo = (g % (128 // L)) * L
        states = idx_v[cb, 0, pl.ds(lo, L)]
        for j in range(NUM_ACTIONS):
            out_v[cb, j, pl.ds(lo, L)] = jnp.take_along_axis(
                cols[j], states, axis=0
            )
        return _

    lax.fori_loop(0, GROUPS, group, 0, unroll=2)

    pltpu.sync_copy(out_v, out_hbm.at[pl.ds(wid * CB_PER_W, CB_PER_W)])


_mesh = plsc.VectorSubcoreMesh(
    core_axis_name="c", subcore_axis_name="s", num_cores=NC, num_subcores=NS
)
_lookup = pl.kernel(
    _body,
    out_type=jax.ShapeDtypeStruct((CBLK, NUM_ACTIONS, 128), jnp.float32),
    mesh=_mesh,
    scratch_types=[
        pltpu.VMEM((CB_PER_W, 1, 128), jnp.int32),
        pltpu.VMEM((NUM_STATES * NUM_ACTIONS,), jnp.float32),
        pltpu.VMEM((CB_PER_W, NUM_ACTIONS, 128), jnp.float32),
        pltpu.SemaphoreType.DMA,
        pltpu.SemaphoreType.DMA,
    ],
    compiler_params=pltpu.CompilerParams(
        needs_layout_passes=False,
        use_tc_tiling_on_sc=False,
        disable_bounds_checks=True,
        disable_semaphore_checks=True,
    ),
)


@jax.jit
def kernel(obs, table):
    obs_view = obs.T.reshape(OBS_DIM, CBLK, 128).transpose(1, 0, 2)
    a = _lookup(obs_view, table.reshape(NUM_STATES * NUM_ACTIONS))
    return a.transpose(1, 0, 2).reshape(NUM_ACTIONS, BATCH).T
